# baseline (speedup 1.0000x reference)
import jax
import jax.numpy as jnp
from jax import lax
from jax.experimental import pallas as pl
from jax.experimental.pallas import tpu as pltpu
from jax.experimental.pallas import tpu_sc as plsc

N = 320000
G = 128
T = 16
S = 256
SCALE = 1000000.0

_INFO = plsc.get_sparse_core_info()
NC = _INFO.num_cores
NS = _INFO.num_subcores
NW = NC * NS
CHUNK = 128
NCHUNKS = N // CHUNK
BASE_PER_W = NCHUNKS // NW
EXTRA = NCHUNKS - BASE_PER_W * NW

RING = 5
DIST = 4
NJ = BASE_PER_W // RING
PEEL = BASE_PER_W - NJ * RING

ROWS_PER_SUB = S // NS
NV = G // 16


def _sc_body(x_hbm, idx_hbm, out_hbm, accum_sh, accum_loc, ibuf_all, ibuf_x,
             zbuf, idbuf, xbufs, g0, g1, g2, g3, g4):
    gsems = (g0, g1, g2, g3, g4)
    c = lax.axis_index("c")
    s = lax.axis_index("s")
    wid = s * NC + c

    fzero16 = jnp.zeros((16,), jnp.float32)
    for r in range(ROWS_PER_SUB):
        for j in range(NV):
            zbuf[r, pl.ds(j * 16, 16)] = fzero16
    pltpu.sync_copy(zbuf, accum_sh.at[pl.ds(s * ROWS_PER_SUB, ROWS_PER_SUB)])
    plsc.subcore_barrier()

    def zrow(r, carry):
        for j in range(NV):
            accum_loc[r, pl.ds(j * 16, 16)] = fzero16
        return carry

    lax.fori_loop(0, S, zrow, 0)

    lane = lax.iota(jnp.int32, 16)
    for h in range(2):
        for j in range(NV):
            idbuf[h, pl.ds(j * 16, 16)] = lane + (128 * h + 16 * j)

    start = wid * BASE_PER_W + jnp.minimum(wid, EXTRA)

    pltpu.sync_copy(idx_hbm.at[pl.ds(start, BASE_PER_W)], ibuf_all)

    @pl.when(wid < EXTRA)
    def _():
        pltpu.sync_copy(idx_hbm.at[start + BASE_PER_W], ibuf_x)

    def issue_gather(t, b):
        pltpu.async_copy(
            x_hbm.at[pl.ds((start + t) * CHUNK, CHUNK)], xbufs.at[b], gsems[b])

    def wait_gather(t, b):
        pltpu.make_async_copy(
            x_hbm.at[pl.ds((start + t) * CHUNK, CHUNK)], xbufs.at[b],
            gsems[b]).wait()

    def process_chunk(t, b, idx_row):
        lo = idx_row[pl.ds(0, 16)]
        hi = lo
        for k in range(1, NV):
            v = idx_row[pl.ds(k * 16, 16)]
            lo = jnp.minimum(lo, v)
            hi = jnp.maximum(hi, v)
        seg_lo = jnp.min(lo)
        seg_hi = jnp.max(hi)

        def local_reduce():
            def rows8(r8, accs):
                accs = list(accs)
                for rr in range(8):
                    for j in range(NV):
                        accs[j] = accs[j] + xbufs[b, r8 * 8 + rr,
                                                 pl.ds(j * 16, 16)]
                return tuple(accs)

            accs = lax.fori_loop(
                0, CHUNK // 8, rows8,
                tuple(jnp.zeros((16,), jnp.float32) for _ in range(NV)))
            for j in range(NV):
                plsc.addupdate(accum_loc.at[seg_lo, pl.ds(j * 16, 16)],
                               accs[j])

        def stream_scatter():
            pltpu.sync_copy(xbufs.at[b], accum_sh.at[idx_row], add=True)

        lax.cond(seg_lo == seg_hi, local_reduce, stream_scatter)

    for b in range(DIST):
        issue_gather(b, b)

    def step(j, carry):
        for b in range(RING):
            t = j * RING + b
            bp = (b + DIST) % RING
            tp = t + DIST

            @pl.when(tp < BASE_PER_W)
            def _(tp=tp, bp=bp):
                issue_gather(tp, bp)

            wait_gather(t, b)
            process_chunk(t, b, ibuf_all.at[t])
        return carry

    lax.fori_loop(0, NJ, step, 0)

    for k in range(PEEL):
        t = NJ * RING + k
        wait_gather(t, t % RING)
        process_chunk(t, t % RING, ibuf_all.at[t])

    @pl.when(wid < EXTRA)
    def _():
        pltpu.sync_copy(
            x_hbm.at[pl.ds((start + BASE_PER_W) * CHUNK, CHUNK)], xbufs.at[0])
        process_chunk(0, 0, ibuf_x)

    pltpu.sync_copy(accum_loc.at[pl.ds(0, 128)], accum_sh.at[idbuf.at[0]],
                    add=True)
    pltpu.sync_copy(accum_loc.at[pl.ds(128, 128)], accum_sh.at[idbuf.at[1]],
                    add=True)
    plsc.subcore_barrier()

    pltpu.sync_copy(
        accum_sh.at[pl.ds(s * ROWS_PER_SUB, ROWS_PER_SUB)],
        out_hbm.at[c, pl.ds(s * ROWS_PER_SUB, ROWS_PER_SUB)],
    )


_sc_segment_sum = pl.kernel(
    _sc_body,
    out_type=jax.ShapeDtypeStruct((NC, S, G), jnp.float32),
    mesh=plsc.VectorSubcoreMesh(core_axis_name="c", subcore_axis_name="s"),
    scratch_types=[
        pltpu.VMEM_SHARED((S, G), jnp.float32),
        pltpu.VMEM((S, G), jnp.float32),
        pltpu.VMEM((BASE_PER_W, CHUNK), jnp.int32),
        pltpu.VMEM((CHUNK,), jnp.int32),
        pltpu.VMEM((ROWS_PER_SUB, G), jnp.float32),
        pltpu.VMEM((2, CHUNK), jnp.int32),
        pltpu.VMEM((RING, CHUNK, G), jnp.float32),
    ] + [pltpu.SemaphoreType.DMA] * RING,
    name="sc_segment_sum",
    compiler_params=pltpu.CompilerParams(use_tc_tiling_on_sc=False,
                                         needs_layout_passes=False),
)


def _tc_body(p_ref, w_ref, ilr_ref, xb_ref):
    xb = p_ref[0] + p_ref[1]
    row_sums = jnp.sum(xb, axis=1, keepdims=True)
    xbn = xb * (SCALE / jnp.maximum(row_sums, 1e-12))
    xb_ref[...] = xbn
    ilr_ref[...] = lax.dot_general(
        xbn, w_ref[...], (((1,), (1,)), ((), ())),
        preferred_element_type=jnp.float32,
    )


def _tc_finish(partials, W):
    return pl.pallas_call(
        _tc_body,
        out_shape=(
            jax.ShapeDtypeStruct((S, T), jnp.float32),
            jax.ShapeDtypeStruct((S, G), jnp.float32),
        ),
    )(partials, W)


@jax.jit
def kernel(X_batch, batch_idx, W):
    idx2d = batch_idx.astype(jnp.int32).reshape(NCHUNKS, CHUNK)
    partials = _sc_segment_sum(X_batch, idx2d)
    ilr_y, X_bulk = _tc_finish(partials, W)
    return (ilr_y, X_bulk)
